# baseline (speedup 1.0000x reference)
import functools

import jax
import jax.numpy as jnp
from jax import lax
from jax.experimental import pallas as pl
from jax.experimental.pallas import tpu as pltpu
from jax.experimental.pallas import tpu_sc as plsc

F32 = jnp.float32


def _mm_body(n_in, relu, *refs):
    xs = refs[:n_in]
    ws = refs[n_in:2 * n_in]
    b = refs[2 * n_in]
    out = refs[2 * n_in + 1]
    acc = None
    for x_ref, w_ref in zip(xs, ws):
        part = lax.dot_general(x_ref[...], w_ref[...],
                               (((1,), (1,)), ((), ())),
                               preferred_element_type=F32)
        acc = part if acc is None else acc + part
    acc = acc + b[...]
    if relu:
        acc = jnp.maximum(acc, 0.0)
    out[...] = acc


def _mm(xs, ws, b, relu=False):
    M = xs[0].shape[0]
    N = ws[0].shape[0]
    ktot = sum(w.shape[1] for w in ws)
    if ktot > 512 or N > 512:
        bm = 1024
    else:
        bm = 4096
    bm = min(M, bm)
    assert M % bm == 0, (M, bm)
    b2 = b.reshape(1, N)
    in_specs = [pl.BlockSpec((bm, x.shape[1]), lambda m: (m, 0)) for x in xs]
    in_specs += [pl.BlockSpec(w.shape, lambda m: (0, 0)) for w in ws]
    in_specs += [pl.BlockSpec((1, N), lambda m: (0, 0))]
    return pl.pallas_call(
        functools.partial(_mm_body, len(xs), relu),
        grid=(M // bm,),
        in_specs=in_specs,
        out_specs=pl.BlockSpec((bm, N), lambda m: (m, 0)),
        out_shape=jax.ShapeDtypeStruct((M, N), F32),
    )(*xs, *ws, b2)


def _dw_body(H, W, *refs):
    x_ref, w_ref, b_ref, out = refs
    bm, C = x_ref.shape
    x = x_ref[...]
    rows = lax.broadcasted_iota(jnp.int32, (bm, 1), 0)
    jw = rows % W
    ih = (rows // W) % H
    acc = jnp.zeros((bm, C), F32) + b_ref[...]
    for oi in (-1, 0, 1):
        if oi == -1:
            mh = (ih >= 1).astype(F32)
        elif oi == 1:
            mh = (ih <= H - 2).astype(F32)
        else:
            mh = None
        for oj in (-1, 0, 1):
            if oj == -1:
                mw = (jw >= 1).astype(F32)
            elif oj == 1:
                mw = (jw <= W - 2).astype(F32)
            else:
                mw = None
            s = oi * W + oj
            if s > 0:
                xs = jnp.concatenate([x[s:], jnp.zeros((s, C), F32)], axis=0)
            elif s < 0:
                xs = jnp.concatenate([jnp.zeros((-s, C), F32), x[:-(-s)]], axis=0)
            else:
                xs = x
            m = None
            if mh is not None and mw is not None:
                m = mh * mw
            elif mh is not None:
                m = mh
            elif mw is not None:
                m = mw
            t = (oi + 1) * 3 + (oj + 1)
            wt = w_ref[t:t + 1, :]
            term = xs * wt
            if m is not None:
                term = term * m
            acc = acc + term
    out[...] = acc


def _dw(x2d, wconv, bias, H, W):
    M, C = x2d.shape
    w9 = wconv.reshape(C, 9).T
    b2 = bias.reshape(1, C)
    imgs_per_blk = max(1, 8192 // (H * W))
    bm = min(M, imgs_per_blk * H * W)
    assert M % bm == 0
    return pl.pallas_call(
        functools.partial(_dw_body, H, W),
        grid=(M // bm,),
        in_specs=[pl.BlockSpec((bm, C), lambda m: (m, 0)),
                  pl.BlockSpec((9, C), lambda m: (0, 0)),
                  pl.BlockSpec((1, C), lambda m: (0, 0))],
        out_specs=pl.BlockSpec((bm, C), lambda m: (m, 0)),
        out_shape=jax.ShapeDtypeStruct((M, C), F32),
    )(x2d, w9, b2)


def _pool_body(x_ref, out):
    out[...] = jnp.maximum(x_ref[:, 0, :], x_ref[:, 1, :])


def _pool_pairs(x3):
    G, _, L = x3.shape
    gb = min(G, max(8, (2 * 1024 * 1024) // (2 * L * 4)) // 8 * 8)
    while G % gb:
        gb //= 2
    return pl.pallas_call(
        _pool_body,
        grid=(G // gb,),
        in_specs=[pl.BlockSpec((gb, 2, L), lambda m: (m, 0, 0))],
        out_specs=pl.BlockSpec((gb, L), lambda m: (m, 0)),
        out_shape=jax.ShapeDtypeStruct((G, L), F32),
    )(x3)


def _maxpool2(x2d, B, H, W, C):
    xh = x2d.reshape(B * (H // 2), 2, W * C)
    x2 = _pool_pairs(xh)
    xw = x2.reshape(B * (H // 2) * (W // 2), 2, C)
    x3 = _pool_pairs(xw)
    return x3


def _c4_body(HW, W, N, nvalid, x_ref, w_ref, b_ref, out):
    kb, _, C = x_ref.shape
    xf = x_ref[...].reshape(kb * HW, C)
    acc = jnp.zeros((kb * HW, N), F32)
    for di in range(3):
        for dj in range(3):
            s = di * W + dj
            t = di * 3 + dj
            if s:
                xs = jnp.concatenate(
                    [xf[s:], jnp.zeros((s, C), F32)], axis=0)
            else:
                xs = xf
            acc = acc + lax.dot_general(
                xs, w_ref[t * C:(t + 1) * C, :], (((1,), (0,)), ((), ())),
                preferred_element_type=F32)
    y = acc.reshape(kb, HW, N)
    rl = lax.broadcasted_iota(jnp.int32, (1, HW, 1), 1)
    m = ((rl // W < HW // W - 2) & (rl % W < W - 2)).astype(F32)
    ssum = jnp.sum(y * m, axis=1)
    out[...] = ssum * (1.0 / nvalid) + b_ref[...]


def _c4_mean(x3, wconv, bias, H, W):
    B, HW, C = x3.shape
    N = wconv.shape[0]
    w9 = jnp.transpose(wconv, (2, 3, 1, 0)).reshape(9 * C, N)
    b2 = bias.reshape(1, N)
    kb = min(B, 64)
    assert B % kb == 0
    nvalid = (H - 2) * (W - 2)
    return pl.pallas_call(
        functools.partial(_c4_body, HW, W, N, nvalid),
        grid=(B // kb,),
        in_specs=[pl.BlockSpec((kb, HW, C), lambda m: (m, 0, 0)),
                  pl.BlockSpec((9 * C, N), lambda m: (0, 0)),
                  pl.BlockSpec((1, N), lambda m: (0, 0))],
        out_specs=pl.BlockSpec((kb, N), lambda m: (m, 0)),
        out_shape=jax.ShapeDtypeStruct((B, N), F32),
    )(x3, w9, b2)


def _head_body(x_ref, w0, b0, w1, b1, out):
    h = lax.dot_general(x_ref[...], w0[...], (((1,), (1,)), ((), ())),
                        preferred_element_type=F32) + b0[...]
    h = jnp.maximum(h, 0.0)
    y = lax.dot_general(h, w1[...], (((1,), (1,)), ((), ())),
                        preferred_element_type=F32) + b1[...]
    y = jnp.maximum(y, 0.0)
    m = jnp.max(y, axis=1, keepdims=True)
    e = jnp.exp(y - m)
    out[...] = (y - m) - jnp.log(jnp.sum(e, axis=1, keepdims=True))


def _head(x, w0, b0, w1, b1):
    M, K = x.shape
    N0 = w0.shape[0]
    N1 = w1.shape[0]
    bm = min(M, 1024)
    return pl.pallas_call(
        _head_body,
        grid=(M // bm,),
        in_specs=[pl.BlockSpec((bm, K), lambda m: (m, 0)),
                  pl.BlockSpec((N0, K), lambda m: (0, 0)),
                  pl.BlockSpec((1, N0), lambda m: (0, 0)),
                  pl.BlockSpec((N1, N0), lambda m: (0, 0)),
                  pl.BlockSpec((1, N1), lambda m: (0, 0))],
        out_specs=pl.BlockSpec((bm, N1), lambda m: (m, 0)),
        out_shape=jax.ShapeDtypeStruct((M, N1), F32),
    )(x, w0, b0.reshape(1, N0), w1, b1.reshape(1, N1))


def _recip_body(p_ref, out):
    c = jnp.sum(jnp.sum(p_ref[...], axis=0), axis=1, keepdims=True)
    out[...] = 1.0 / jnp.maximum(c, 1.0)


def _recip_counts(partial):
    W, V, L = partial.shape
    return pl.pallas_call(
        _recip_body,
        in_specs=[pl.BlockSpec((W, V, L), lambda: (0, 0, 0))],
        out_specs=pl.BlockSpec((V, 1), lambda: (0, 0)),
        out_shape=jax.ShapeDtypeStruct((V, 1), F32),
    )(partial)


def _combine_body(p_ref, r_ref, out):
    r = r_ref[...]
    parts = [(p_ref[0, ct] + p_ref[1, ct]) * r for ct in range(p_ref.shape[1])]
    out[...] = jnp.concatenate(parts, axis=1)


def _combine_mean(partial, recip):
    _, CT, V, dsub = partial.shape
    return pl.pallas_call(
        _combine_body,
        in_specs=[pl.BlockSpec((2, CT, V, dsub), lambda: (0, 0, 0, 0)),
                  pl.BlockSpec((V, 1), lambda: (0, 0))],
        out_specs=pl.BlockSpec((V, CT * dsub), lambda: (0, 0)),
        out_shape=jax.ShapeDtypeStruct((V, CT * dsub), F32),
    )(partial, recip)


_NW = 32


def _sc_gather(table, idx):
    V, D = table.shape
    B = idx.shape[0]
    bpw = B // _NW
    ch = min(128, bpw, max(16, 65536 // D))
    n_ch = bpw // ch
    assert bpw % ch == 0
    mesh = plsc.VectorSubcoreMesh(core_axis_name="c", subcore_axis_name="s")

    @functools.partial(
        pl.kernel, mesh=mesh,
        out_type=jax.ShapeDtypeStruct((B, D), F32),
        scratch_types=[pltpu.VMEM((ch,), jnp.int32),
                       pltpu.VMEM((ch, D), F32),
                       pltpu.SemaphoreType.DMA],
    )
    def k(table_hbm, idx_hbm, out_hbm, idx_v, rows_v, sem):
        wid = lax.axis_index("s") * 2 + lax.axis_index("c")
        base = wid * bpw
        for j in range(n_ch):
            off = base + j * ch
            pltpu.sync_copy(idx_hbm.at[pl.ds(off, ch)], idx_v)
            pltpu.async_copy(table_hbm.at[idx_v], rows_v, sem).wait()
            pltpu.sync_copy(rows_v, out_hbm.at[pl.ds(off, ch)])

    return k(table, idx)


def _sc_scatter_add(vals_t, idx, V):
    D, E = vals_t.shape
    dsub = D // 16
    assert dsub >= 16 and dsub % 16 == 0
    eh = E // 2
    ch_e = 128
    n_ch = eh // ch_e
    zeros = jnp.zeros((V * dsub,), F32)
    mesh = plsc.VectorSubcoreMesh(core_axis_name="c", subcore_axis_name="s")

    @functools.partial(
        pl.kernel, mesh=mesh,
        out_type=jax.ShapeDtypeStruct((2, 16, V * dsub), F32),
        scratch_types=[pltpu.VMEM((ch_e,), jnp.int32),
                       pltpu.VMEM((dsub, ch_e), F32),
                       pltpu.VMEM((V * dsub,), F32)],
        compiler_params=pltpu.CompilerParams(needs_layout_passes=False),
    )
    def k(vals_hbm, idx_hbm, zeros_hbm, out_hbm, idx_v, val_v, acc_v):
        c = lax.axis_index("c")
        s = lax.axis_index("s")
        wid = s * 2 + c
        rep = wid // 16
        ct = wid % 16
        pltpu.sync_copy(zeros_hbm, acc_v)
        lanes = lax.iota(jnp.int32, 16)
        for j in range(n_ch):
            off = rep * eh + j * ch_e
            pltpu.sync_copy(idx_hbm.at[pl.ds(off, ch_e)], idx_v)
            pltpu.sync_copy(
                vals_hbm.at[pl.ds(ct * dsub, dsub), pl.ds(off, ch_e)], val_v)

            def body(eg, _):
                rvec = idx_v[pl.ds(eg * 16, 16)]
                for l in range(16):
                    e = eg * 16 + l
                    rows = jnp.full((16,), rvec[l], jnp.int32)
                    es = jnp.full((16,), e, jnp.int32)
                    for g in range(dsub // 16):
                        v = plsc.load_gather(val_v, [g * 16 + lanes, es])
                        plsc.addupdate_scatter(
                            acc_v, [rows * dsub + g * 16 + lanes], v)
                return 0

            lax.fori_loop(0, ch_e // 16, body, 0)
        pltpu.sync_copy(acc_v, out_hbm.at[rep, ct])

    return k(vals_t, idx, zeros)


def _sc_counts(idx, E, V):
    epw = E // _NW
    zeros = jnp.zeros((V * 16,), F32)
    mesh = plsc.VectorSubcoreMesh(core_axis_name="c", subcore_axis_name="s")

    @functools.partial(
        pl.kernel, mesh=mesh,
        out_type=jax.ShapeDtypeStruct((_NW, V * 16), F32),
        scratch_types=[pltpu.VMEM((epw,), jnp.int32),
                       pltpu.VMEM((V * 16,), F32)],
        compiler_params=pltpu.CompilerParams(needs_layout_passes=False),
    )
    def k(idx_hbm, zeros_hbm, out_hbm, idx_v, acc_v):
        c = lax.axis_index("c")
        s = lax.axis_index("s")
        wid = s * 2 + c
        pltpu.sync_copy(zeros_hbm, acc_v)
        pltpu.sync_copy(idx_hbm.at[pl.ds(wid * epw, epw)], idx_v)
        lanes = lax.iota(jnp.int32, 16)
        one = jnp.ones((16,), F32)

        def body(g, _):
            rows = idx_v[pl.ds(g * 16, 16)]
            plsc.addupdate_scatter(acc_v, [rows * 16 + lanes], one)
            return 0

        lax.fori_loop(0, epw // 16, body, 0)
        pltpu.sync_copy(acc_v, out_hbm.at[wid])

    return k(idx, zeros)


def _cnn(p, imgs, c_mid, n_pools):
    B = imgs.shape[0]
    x = jnp.transpose(imgs, (0, 2, 3, 1)).reshape(B * 256, 3)
    x = _dw(x, p["dw1"][0], p["dw1"][1], 16, 16)
    x = _mm([x], [p["pw1"][0].reshape(c_mid, 3)], p["pw1"][1])
    x = _dw(x, p["dw2"][0], p["dw2"][1], 16, 16)
    x = _mm([x], [p["pw2"][0].reshape(c_mid, c_mid)], p["pw2"][1])
    x = _maxpool2(x, B, 16, 16, c_mid)
    x = _dw(x, p["dw3"][0], p["dw3"][1], 8, 8)
    c_out = p["pw3"][0].shape[0]
    x = _mm([x], [p["pw3"][0].reshape(c_out, c_mid)], p["pw3"][1])
    hw = 8
    if n_pools == 2:
        x = _maxpool2(x, B, 8, 8, c_out)
        hw = 4
    x = _c4_mean(x.reshape(B, hw * hw, c_out), p["c4"][0], p["c4"][1], hw, hw)
    return x


def _split_cols(W, widths):
    out = []
    o = 0
    for w in widths:
        out.append(W[:, o:o + w])
        o += w
    return out


def kernel(x, edge_attr, node_image_regions, edge_image_regions, edge_index,
           params):
    p = params
    row = edge_index[0].astype(jnp.int32)
    idx_all = edge_index.astype(jnp.int32).reshape(-1)
    n_nodes = x.shape[0]
    n_edges = edge_attr.shape[0]

    nv = _cnn(p["node_cnn"], node_image_regions, 64, 1)
    ev = _cnn(p["edge_cnn"], edge_image_regions, 128, 2)

    wj = _split_cols(p["node_join"][0], [x.shape[1], 256])
    xc = _mm([x, nv], wj, p["node_join"][1])
    wj = _split_cols(p["edge_join"][0], [edge_attr.shape[1], 256])
    ea = _mm([edge_attr, ev], wj, p["edge_join"][1])

    cnt_partial = _sc_counts(row, n_edges, n_nodes).reshape(_NW, n_nodes, 16)
    recip = _recip_counts(cnt_partial)

    dims = [(256, 256, 256, 512, 512), (512, 512, 512, 1024, 1024),
            (1024, 1024, 1024, 512, 512), (512, 512, 512, 256, 256)]
    for i, (inn, ine, hid, outn, oute) in enumerate(dims):
        pe = p["l%d_edge" % (i + 1)]
        pn = p["l%d_node" % (i + 1)]
        g = _sc_gather(xc, idx_all)
        src, dst = g[:n_edges], g[n_edges:]
        ws = _split_cols(pe["mlp0"][0], [inn, inn, ine])
        h = _mm([src, dst, ea], ws, pe["mlp0"][1], relu=True)
        h = _mm([h], [pe["mlp1"][0]], pe["mlp1"][1])
        ws = _split_cols(pe["res"][0], [oute, ine])
        ea = _mm([h, ea], ws, pe["res"][1])
        ws = _split_cols(pn["mlp1_0"][0], [inn, oute])
        h = _mm([dst, ea], ws, pn["mlp1_0"][1], relu=True)
        h = _mm([h], [pn["mlp1_1"][0]], pn["mlp1_1"][1])
        part = _sc_scatter_add(h.T, row, n_nodes)
        part = part.reshape(2, 16, n_nodes, outn // 16)
        agg = _combine_mean(part, recip)
        ws = _split_cols(pn["mlp2_0"][0], [inn, outn])
        h = _mm([xc, agg], ws, pn["mlp2_0"][1], relu=True)
        h = _mm([h], [pn["mlp2_1"][0]], pn["mlp2_1"][1])
        ws = _split_cols(pn["res"][0], [outn, inn])
        xc = _mm([h, xc], ws, pn["res"][1])

    xn = _head(xc, p["node_cls0"][0], p["node_cls0"][1],
               p["node_cls1"][0], p["node_cls1"][1])
    xe = _head(ea, p["edge_cls0"][0], p["edge_cls0"][1],
               p["edge_cls1"][0], p["edge_cls1"][1])
    return (xn, xe)


# speedup vs baseline: 1.0047x; 1.0047x over previous
import functools

import jax
import jax.numpy as jnp
from jax import lax
from jax.experimental import pallas as pl
from jax.experimental.pallas import tpu as pltpu
from jax.experimental.pallas import tpu_sc as plsc

F32 = jnp.float32


def _mm_body(n_in, relu, t_out, *refs):
    xs = refs[:n_in]
    ws = refs[n_in:2 * n_in]
    b = refs[2 * n_in]
    out = refs[2 * n_in + 1]
    acc = None
    for x_ref, w_ref in zip(xs, ws):
        if t_out:
            part = lax.dot_general(w_ref[...], x_ref[...],
                                   (((1,), (1,)), ((), ())),
                                   preferred_element_type=F32)
        else:
            part = lax.dot_general(x_ref[...], w_ref[...],
                                   (((1,), (1,)), ((), ())),
                                   preferred_element_type=F32)
        acc = part if acc is None else acc + part
    acc = acc + b[...]
    if relu:
        acc = jnp.maximum(acc, 0.0)
    out[...] = acc


def _mm(xs, ws, b, relu=False, t_out=False):
    M = xs[0].shape[0]
    N = ws[0].shape[0]
    ktot = sum(w.shape[1] for w in ws)
    if ktot > 512 or N > 512:
        bm = 1024
    else:
        bm = 4096
    bm = min(M, bm)
    assert M % bm == 0, (M, bm)
    in_specs = [pl.BlockSpec((bm, x.shape[1]), lambda m: (m, 0)) for x in xs]
    in_specs += [pl.BlockSpec(w.shape, lambda m: (0, 0)) for w in ws]
    if t_out:
        b2 = b.reshape(N, 1)
        in_specs += [pl.BlockSpec((N, 1), lambda m: (0, 0))]
        out_specs = pl.BlockSpec((N, bm), lambda m: (0, m))
        out_shape = jax.ShapeDtypeStruct((N, M), F32)
    else:
        b2 = b.reshape(1, N)
        in_specs += [pl.BlockSpec((1, N), lambda m: (0, 0))]
        out_specs = pl.BlockSpec((bm, N), lambda m: (m, 0))
        out_shape = jax.ShapeDtypeStruct((M, N), F32)
    return pl.pallas_call(
        functools.partial(_mm_body, len(xs), relu, t_out),
        grid=(M // bm,),
        in_specs=in_specs,
        out_specs=out_specs,
        out_shape=out_shape,
    )(*xs, *ws, b2)


def _dw_body(H, W, *refs):
    x_ref, w_ref, b_ref, out = refs
    bm, C = x_ref.shape
    x = x_ref[...]
    rows = lax.broadcasted_iota(jnp.int32, (bm, 1), 0)
    jw = rows % W
    ih = (rows // W) % H
    acc = jnp.zeros((bm, C), F32) + b_ref[...]
    for oi in (-1, 0, 1):
        if oi == -1:
            mh = (ih >= 1).astype(F32)
        elif oi == 1:
            mh = (ih <= H - 2).astype(F32)
        else:
            mh = None
        for oj in (-1, 0, 1):
            if oj == -1:
                mw = (jw >= 1).astype(F32)
            elif oj == 1:
                mw = (jw <= W - 2).astype(F32)
            else:
                mw = None
            s = oi * W + oj
            if s > 0:
                xs = jnp.concatenate([x[s:], jnp.zeros((s, C), F32)], axis=0)
            elif s < 0:
                xs = jnp.concatenate([jnp.zeros((-s, C), F32), x[:-(-s)]], axis=0)
            else:
                xs = x
            m = None
            if mh is not None and mw is not None:
                m = mh * mw
            elif mh is not None:
                m = mh
            elif mw is not None:
                m = mw
            t = (oi + 1) * 3 + (oj + 1)
            wt = w_ref[t:t + 1, :]
            term = xs * wt
            if m is not None:
                term = term * m
            acc = acc + term
    out[...] = acc


def _dw(x2d, wconv, bias, H, W):
    M, C = x2d.shape
    w9 = wconv.reshape(C, 9).T
    b2 = bias.reshape(1, C)
    imgs_per_blk = max(1, 8192 // (H * W))
    bm = min(M, imgs_per_blk * H * W)
    assert M % bm == 0
    return pl.pallas_call(
        functools.partial(_dw_body, H, W),
        grid=(M // bm,),
        in_specs=[pl.BlockSpec((bm, C), lambda m: (m, 0)),
                  pl.BlockSpec((9, C), lambda m: (0, 0)),
                  pl.BlockSpec((1, C), lambda m: (0, 0))],
        out_specs=pl.BlockSpec((bm, C), lambda m: (m, 0)),
        out_shape=jax.ShapeDtypeStruct((M, C), F32),
    )(x2d, w9, b2)


def _pool_body(x_ref, out):
    out[...] = jnp.maximum(x_ref[:, 0, :], x_ref[:, 1, :])


def _pool_pairs(x3):
    G, _, L = x3.shape
    gb = min(G, max(8, (2 * 1024 * 1024) // (2 * L * 4)) // 8 * 8)
    while G % gb:
        gb //= 2
    return pl.pallas_call(
        _pool_body,
        grid=(G // gb,),
        in_specs=[pl.BlockSpec((gb, 2, L), lambda m: (m, 0, 0))],
        out_specs=pl.BlockSpec((gb, L), lambda m: (m, 0)),
        out_shape=jax.ShapeDtypeStruct((G, L), F32),
    )(x3)


def _maxpool2(x2d, B, H, W, C):
    xh = x2d.reshape(B * (H // 2), 2, W * C)
    x2 = _pool_pairs(xh)
    xw = x2.reshape(B * (H // 2) * (W // 2), 2, C)
    x3 = _pool_pairs(xw)
    return x3


def _c4_body(HW, W, N, nvalid, x_ref, w_ref, b_ref, out):
    kb, _, C = x_ref.shape
    xf = x_ref[...].reshape(kb * HW, C)
    acc = jnp.zeros((kb * HW, N), F32)
    for di in range(3):
        for dj in range(3):
            s = di * W + dj
            t = di * 3 + dj
            if s:
                xs = jnp.concatenate(
                    [xf[s:], jnp.zeros((s, C), F32)], axis=0)
            else:
                xs = xf
            acc = acc + lax.dot_general(
                xs, w_ref[t * C:(t + 1) * C, :], (((1,), (0,)), ((), ())),
                preferred_element_type=F32)
    y = acc.reshape(kb, HW, N)
    rl = lax.broadcasted_iota(jnp.int32, (1, HW, 1), 1)
    m = ((rl // W < HW // W - 2) & (rl % W < W - 2)).astype(F32)
    ssum = jnp.sum(y * m, axis=1)
    out[...] = ssum * (1.0 / nvalid) + b_ref[...]


def _c4_mean(x3, wconv, bias, H, W):
    B, HW, C = x3.shape
    N = wconv.shape[0]
    w9 = jnp.transpose(wconv, (2, 3, 1, 0)).reshape(9 * C, N)
    b2 = bias.reshape(1, N)
    kb = min(B, 64)
    assert B % kb == 0
    nvalid = (H - 2) * (W - 2)
    return pl.pallas_call(
        functools.partial(_c4_body, HW, W, N, nvalid),
        grid=(B // kb,),
        in_specs=[pl.BlockSpec((kb, HW, C), lambda m: (m, 0, 0)),
                  pl.BlockSpec((9 * C, N), lambda m: (0, 0)),
                  pl.BlockSpec((1, N), lambda m: (0, 0))],
        out_specs=pl.BlockSpec((kb, N), lambda m: (m, 0)),
        out_shape=jax.ShapeDtypeStruct((B, N), F32),
    )(x3, w9, b2)


def _head_body(x_ref, w0, b0, w1, b1, out):
    h = lax.dot_general(x_ref[...], w0[...], (((1,), (1,)), ((), ())),
                        preferred_element_type=F32) + b0[...]
    h = jnp.maximum(h, 0.0)
    y = lax.dot_general(h, w1[...], (((1,), (1,)), ((), ())),
                        preferred_element_type=F32) + b1[...]
    y = jnp.maximum(y, 0.0)
    m = jnp.max(y, axis=1, keepdims=True)
    e = jnp.exp(y - m)
    out[...] = (y - m) - jnp.log(jnp.sum(e, axis=1, keepdims=True))


def _head(x, w0, b0, w1, b1):
    M, K = x.shape
    N0 = w0.shape[0]
    N1 = w1.shape[0]
    bm = min(M, 1024)
    return pl.pallas_call(
        _head_body,
        grid=(M // bm,),
        in_specs=[pl.BlockSpec((bm, K), lambda m: (m, 0)),
                  pl.BlockSpec((N0, K), lambda m: (0, 0)),
                  pl.BlockSpec((1, N0), lambda m: (0, 0)),
                  pl.BlockSpec((N1, N0), lambda m: (0, 0)),
                  pl.BlockSpec((1, N1), lambda m: (0, 0))],
        out_specs=pl.BlockSpec((bm, N1), lambda m: (m, 0)),
        out_shape=jax.ShapeDtypeStruct((M, N1), F32),
    )(x, w0, b0.reshape(1, N0), w1, b1.reshape(1, N1))


def _recip_body(p_ref, out):
    c = jnp.sum(jnp.sum(p_ref[...], axis=0), axis=1, keepdims=True)
    out[...] = 1.0 / jnp.maximum(c, 1.0)


def _recip_counts(partial):
    W, V, L = partial.shape
    return pl.pallas_call(
        _recip_body,
        in_specs=[pl.BlockSpec((W, V, L), lambda: (0, 0, 0))],
        out_specs=pl.BlockSpec((V, 1), lambda: (0, 0)),
        out_shape=jax.ShapeDtypeStruct((V, 1), F32),
    )(partial)


def _combine_body(p_ref, r_ref, out):
    r = r_ref[...]
    parts = [(p_ref[0, ct] + p_ref[1, ct]) * r for ct in range(p_ref.shape[1])]
    out[...] = jnp.concatenate(parts, axis=1)


def _combine_mean(partial, recip):
    _, CT, V, dsub = partial.shape
    return pl.pallas_call(
        _combine_body,
        in_specs=[pl.BlockSpec((2, CT, V, dsub), lambda: (0, 0, 0, 0)),
                  pl.BlockSpec((V, 1), lambda: (0, 0))],
        out_specs=pl.BlockSpec((V, CT * dsub), lambda: (0, 0)),
        out_shape=jax.ShapeDtypeStruct((V, CT * dsub), F32),
    )(partial, recip)


_NW = 32


def _sc_gather(table, idx):
    V, D = table.shape
    B = idx.shape[0]
    bpw = B // _NW
    ch = min(128, bpw, max(16, 65536 // D))
    n_ch = bpw // ch
    assert bpw % ch == 0
    mesh = plsc.VectorSubcoreMesh(core_axis_name="c", subcore_axis_name="s")

    @functools.partial(
        pl.kernel, mesh=mesh,
        out_type=jax.ShapeDtypeStruct((B, D), F32),
        scratch_types=[pltpu.VMEM((ch,), jnp.int32),
                       pltpu.VMEM((ch, D), F32),
                       pltpu.SemaphoreType.DMA],
    )
    def k(table_hbm, idx_hbm, out_hbm, idx_v, rows_v, sem):
        wid = lax.axis_index("s") * 2 + lax.axis_index("c")
        base = wid * bpw
        for j in range(n_ch):
            off = base + j * ch
            pltpu.sync_copy(idx_hbm.at[pl.ds(off, ch)], idx_v)
            pltpu.async_copy(table_hbm.at[idx_v], rows_v, sem).wait()
            pltpu.sync_copy(rows_v, out_hbm.at[pl.ds(off, ch)])

    return k(table, idx)


def _sc_scatter_add(vals_t, idx, V):
    D, E = vals_t.shape
    dsub = D // 16
    assert dsub >= 16 and dsub % 16 == 0
    eh = E // 2
    ch_e = 128
    n_ch = eh // ch_e
    zeros = jnp.zeros((V * dsub,), F32)
    mesh = plsc.VectorSubcoreMesh(core_axis_name="c", subcore_axis_name="s")

    @functools.partial(
        pl.kernel, mesh=mesh,
        out_type=jax.ShapeDtypeStruct((2, 16, V * dsub), F32),
        scratch_types=[pltpu.VMEM((ch_e,), jnp.int32),
                       pltpu.VMEM((dsub, ch_e), F32),
                       pltpu.VMEM((V * dsub,), F32)],
        compiler_params=pltpu.CompilerParams(needs_layout_passes=False),
    )
    def k(vals_hbm, idx_hbm, zeros_hbm, out_hbm, idx_v, val_v, acc_v):
        c = lax.axis_index("c")
        s = lax.axis_index("s")
        wid = s * 2 + c
        rep = wid // 16
        ct = wid % 16
        pltpu.sync_copy(zeros_hbm, acc_v)
        lanes = lax.iota(jnp.int32, 16)
        for j in range(n_ch):
            off = rep * eh + j * ch_e
            pltpu.sync_copy(idx_hbm.at[pl.ds(off, ch_e)], idx_v)
            pltpu.sync_copy(
                vals_hbm.at[pl.ds(ct * dsub, dsub), pl.ds(off, ch_e)], val_v)

            def body(eg, _):
                rvec = idx_v[pl.ds(eg * 16, 16)]
                for l in range(16):
                    e = eg * 16 + l
                    rows = jnp.full((16,), rvec[l], jnp.int32)
                    es = jnp.full((16,), e, jnp.int32)
                    for g in range(dsub // 16):
                        v = plsc.load_gather(val_v, [g * 16 + lanes, es])
                        plsc.addupdate_scatter(
                            acc_v, [rows * dsub + g * 16 + lanes], v)
                return 0

            lax.fori_loop(0, ch_e // 16, body, 0)
        pltpu.sync_copy(acc_v, out_hbm.at[rep, ct])

    return k(vals_t, idx, zeros)


def _sc_counts(idx, E, V):
    epw = E // _NW
    zeros = jnp.zeros((V * 16,), F32)
    mesh = plsc.VectorSubcoreMesh(core_axis_name="c", subcore_axis_name="s")

    @functools.partial(
        pl.kernel, mesh=mesh,
        out_type=jax.ShapeDtypeStruct((_NW, V * 16), F32),
        scratch_types=[pltpu.VMEM((epw,), jnp.int32),
                       pltpu.VMEM((V * 16,), F32)],
        compiler_params=pltpu.CompilerParams(needs_layout_passes=False),
    )
    def k(idx_hbm, zeros_hbm, out_hbm, idx_v, acc_v):
        c = lax.axis_index("c")
        s = lax.axis_index("s")
        wid = s * 2 + c
        pltpu.sync_copy(zeros_hbm, acc_v)
        pltpu.sync_copy(idx_hbm.at[pl.ds(wid * epw, epw)], idx_v)
        lanes = lax.iota(jnp.int32, 16)
        one = jnp.ones((16,), F32)

        def body(g, _):
            rows = idx_v[pl.ds(g * 16, 16)]
            plsc.addupdate_scatter(acc_v, [rows * 16 + lanes], one)
            return 0

        lax.fori_loop(0, epw // 16, body, 0)
        pltpu.sync_copy(acc_v, out_hbm.at[wid])

    return k(idx, zeros)


def _cnn(p, imgs, c_mid, n_pools):
    B = imgs.shape[0]
    x = jnp.transpose(imgs, (0, 2, 3, 1)).reshape(B * 256, 3)
    x = _dw(x, p["dw1"][0], p["dw1"][1], 16, 16)
    x = _mm([x], [p["pw1"][0].reshape(c_mid, 3)], p["pw1"][1])
    x = _dw(x, p["dw2"][0], p["dw2"][1], 16, 16)
    x = _mm([x], [p["pw2"][0].reshape(c_mid, c_mid)], p["pw2"][1])
    x = _maxpool2(x, B, 16, 16, c_mid)
    x = _dw(x, p["dw3"][0], p["dw3"][1], 8, 8)
    c_out = p["pw3"][0].shape[0]
    x = _mm([x], [p["pw3"][0].reshape(c_out, c_mid)], p["pw3"][1])
    hw = 8
    if n_pools == 2:
        x = _maxpool2(x, B, 8, 8, c_out)
        hw = 4
    x = _c4_mean(x.reshape(B, hw * hw, c_out), p["c4"][0], p["c4"][1], hw, hw)
    return x


def _split_cols(W, widths):
    out = []
    o = 0
    for w in widths:
        out.append(W[:, o:o + w])
        o += w
    return out


def kernel(x, edge_attr, node_image_regions, edge_image_regions, edge_index,
           params):
    p = params
    row = edge_index[0].astype(jnp.int32)
    idx_all = edge_index.astype(jnp.int32).reshape(-1)
    n_nodes = x.shape[0]
    n_edges = edge_attr.shape[0]

    nv = _cnn(p["node_cnn"], node_image_regions, 64, 1)
    ev = _cnn(p["edge_cnn"], edge_image_regions, 128, 2)

    wj = _split_cols(p["node_join"][0], [x.shape[1], 256])
    xc = _mm([x, nv], wj, p["node_join"][1])
    wj = _split_cols(p["edge_join"][0], [edge_attr.shape[1], 256])
    ea = _mm([edge_attr, ev], wj, p["edge_join"][1])

    cnt_partial = _sc_counts(row, n_edges, n_nodes).reshape(_NW, n_nodes, 16)
    recip = _recip_counts(cnt_partial)

    dims = [(256, 256, 256, 512, 512), (512, 512, 512, 1024, 1024),
            (1024, 1024, 1024, 512, 512), (512, 512, 512, 256, 256)]
    for i, (inn, ine, hid, outn, oute) in enumerate(dims):
        pe = p["l%d_edge" % (i + 1)]
        pn = p["l%d_node" % (i + 1)]
        g = _sc_gather(xc, idx_all)
        src, dst = g[:n_edges], g[n_edges:]
        ws = _split_cols(pe["mlp0"][0], [inn, inn, ine])
        h = _mm([src, dst, ea], ws, pe["mlp0"][1], relu=True)
        h = _mm([h], [pe["mlp1"][0]], pe["mlp1"][1])
        ws = _split_cols(pe["res"][0], [oute, ine])
        ea = _mm([h, ea], ws, pe["res"][1])
        ws = _split_cols(pn["mlp1_0"][0], [inn, oute])
        h = _mm([dst, ea], ws, pn["mlp1_0"][1], relu=True)
        ht = _mm([h], [pn["mlp1_1"][0]], pn["mlp1_1"][1], t_out=True)
        part = _sc_scatter_add(ht, row, n_nodes)
        part = part.reshape(2, 16, n_nodes, outn // 16)
        agg = _combine_mean(part, recip)
        ws = _split_cols(pn["mlp2_0"][0], [inn, outn])
        h = _mm([xc, agg], ws, pn["mlp2_0"][1], relu=True)
        h = _mm([h], [pn["mlp2_1"][0]], pn["mlp2_1"][1])
        ws = _split_cols(pn["res"][0], [outn, inn])
        xc = _mm([h, xc], ws, pn["res"][1])

    xn = _head(xc, p["node_cls0"][0], p["node_cls0"][1],
               p["node_cls1"][0], p["node_cls1"][1])
    xe = _head(ea, p["edge_cls0"][0], p["edge_cls0"][1],
               p["edge_cls1"][0], p["edge_cls1"][1])
    return (xn, xe)


# speedup vs baseline: 1.5695x; 1.5622x over previous
import functools

import jax
import jax.numpy as jnp
from jax import lax
from jax.experimental import pallas as pl
from jax.experimental.pallas import tpu as pltpu
from jax.experimental.pallas import tpu_sc as plsc

F32 = jnp.float32


def _mm_body(n_in, relu, t_out, *refs):
    xs = refs[:n_in]
    ws = refs[n_in:2 * n_in]
    b = refs[2 * n_in]
    out = refs[2 * n_in + 1]
    acc = None
    for x_ref, w_ref in zip(xs, ws):
        if t_out:
            part = lax.dot_general(w_ref[...], x_ref[...],
                                   (((1,), (1,)), ((), ())),
                                   preferred_element_type=F32)
        else:
            part = lax.dot_general(x_ref[...], w_ref[...],
                                   (((1,), (1,)), ((), ())),
                                   preferred_element_type=F32)
        acc = part if acc is None else acc + part
    acc = acc + b[...]
    if relu:
        acc = jnp.maximum(acc, 0.0)
    out[...] = acc


def _mm(xs, ws, b, relu=False, t_out=False):
    M = xs[0].shape[0]
    N = ws[0].shape[0]
    ktot = sum(w.shape[1] for w in ws)
    if ktot > 512 or N > 512:
        bm = 1024
    else:
        bm = 4096
    bm = min(M, bm)
    assert M % bm == 0, (M, bm)
    in_specs = [pl.BlockSpec((bm, x.shape[1]), lambda m: (m, 0)) for x in xs]
    in_specs += [pl.BlockSpec(w.shape, lambda m: (0, 0)) for w in ws]
    if t_out:
        b2 = b.reshape(N, 1)
        in_specs += [pl.BlockSpec((N, 1), lambda m: (0, 0))]
        out_specs = pl.BlockSpec((N, bm), lambda m: (0, m))
        out_shape = jax.ShapeDtypeStruct((N, M), F32)
    else:
        b2 = b.reshape(1, N)
        in_specs += [pl.BlockSpec((1, N), lambda m: (0, 0))]
        out_specs = pl.BlockSpec((bm, N), lambda m: (m, 0))
        out_shape = jax.ShapeDtypeStruct((M, N), F32)
    return pl.pallas_call(
        functools.partial(_mm_body, len(xs), relu, t_out),
        grid=(M // bm,),
        in_specs=in_specs,
        out_specs=out_specs,
        out_shape=out_shape,
    )(*xs, *ws, b2)


def _dwpw1_body(Cout, x_ref, w9_ref, b1_ref, w1_ref, b2_ref, out_ref):
    kb = x_ref.shape[0]
    x = x_ref[...]
    hw = lax.broadcasted_iota(jnp.int32, (1, 1, 256), 2)
    jw = hw % 16
    ih = hw // 16
    acc = jnp.zeros((kb, 3, 256), F32) + b1_ref[...][None]
    for oi in (-1, 0, 1):
        mh = None if oi == 0 else (
            (ih >= 1) if oi < 0 else (ih <= 14)).astype(F32)
        for oj in (-1, 0, 1):
            mw = None if oj == 0 else (
                (jw >= 1) if oj < 0 else (jw <= 14)).astype(F32)
            s = oi * 16 + oj
            if s > 0:
                xs = jnp.concatenate(
                    [x[:, :, s:], jnp.zeros((kb, 3, s), F32)], axis=2)
            elif s < 0:
                xs = jnp.concatenate(
                    [jnp.zeros((kb, 3, -s), F32), x[:, :, :s]], axis=2)
            else:
                xs = x
            t = (oi + 1) * 3 + (oj + 1)
            wt = w9_ref[:, t:t + 1][None]
            term = xs * wt
            if mh is not None:
                term = term * mh
            if mw is not None:
                term = term * mw
            acc = acc + term
    z = jnp.zeros((kb, Cout, 256), F32) + b2_ref[...][None]
    for c in range(3):
        z = z + acc[:, c:c + 1, :] * w1_ref[:, c:c + 1][None]
    out_ref[...] = lax.transpose(z, (0, 2, 1))


def _dwpw1(imgs, dw_wb, pw_wb, Cout):
    B = imgs.shape[0]
    x3 = imgs.reshape(B, 3, 256)
    w9 = dw_wb[0].reshape(3, 9)
    b1 = dw_wb[1].reshape(3, 1)
    w1 = pw_wb[0].reshape(Cout, 3)
    b2 = pw_wb[1].reshape(Cout, 1)
    kb = min(B, 4 * 1024 * 1024 // (Cout * 256 * 4))
    while B % kb:
        kb -= 1
    out = pl.pallas_call(
        functools.partial(_dwpw1_body, Cout),
        grid=(B // kb,),
        in_specs=[pl.BlockSpec((kb, 3, 256), lambda m: (m, 0, 0)),
                  pl.BlockSpec((3, 9), lambda m: (0, 0)),
                  pl.BlockSpec((3, 1), lambda m: (0, 0)),
                  pl.BlockSpec((Cout, 3), lambda m: (0, 0)),
                  pl.BlockSpec((Cout, 1), lambda m: (0, 0))],
        out_specs=pl.BlockSpec((kb, 256, Cout), lambda m: (m, 0, 0)),
        out_shape=jax.ShapeDtypeStruct((B, 256, Cout), F32),
    )(x3, w9, b1, w1, b2)
    return out.reshape(B * 256, Cout)


def _dw_body(H, W, *refs):
    x_ref, w_ref, b_ref, out = refs
    bm, C = x_ref.shape
    x = x_ref[...]
    rows = lax.broadcasted_iota(jnp.int32, (bm, 1), 0)
    jw = rows % W
    ih = (rows // W) % H
    acc = jnp.zeros((bm, C), F32) + b_ref[...]
    for oi in (-1, 0, 1):
        if oi == -1:
            mh = (ih >= 1).astype(F32)
        elif oi == 1:
            mh = (ih <= H - 2).astype(F32)
        else:
            mh = None
        for oj in (-1, 0, 1):
            if oj == -1:
                mw = (jw >= 1).astype(F32)
            elif oj == 1:
                mw = (jw <= W - 2).astype(F32)
            else:
                mw = None
            s = oi * W + oj
            if s > 0:
                xs = jnp.concatenate([x[s:], jnp.zeros((s, C), F32)], axis=0)
            elif s < 0:
                xs = jnp.concatenate([jnp.zeros((-s, C), F32), x[:-(-s)]], axis=0)
            else:
                xs = x
            m = None
            if mh is not None and mw is not None:
                m = mh * mw
            elif mh is not None:
                m = mh
            elif mw is not None:
                m = mw
            t = (oi + 1) * 3 + (oj + 1)
            wt = w_ref[t:t + 1, :]
            term = xs * wt
            if m is not None:
                term = term * m
            acc = acc + term
    out[...] = acc


def _dw(x2d, wconv, bias, H, W):
    M, C = x2d.shape
    w9 = wconv.reshape(C, 9).T
    b2 = bias.reshape(1, C)
    imgs_per_blk = max(1, 8192 // (H * W))
    bm = min(M, imgs_per_blk * H * W)
    assert M % bm == 0
    return pl.pallas_call(
        functools.partial(_dw_body, H, W),
        grid=(M // bm,),
        in_specs=[pl.BlockSpec((bm, C), lambda m: (m, 0)),
                  pl.BlockSpec((9, C), lambda m: (0, 0)),
                  pl.BlockSpec((1, C), lambda m: (0, 0))],
        out_specs=pl.BlockSpec((bm, C), lambda m: (m, 0)),
        out_shape=jax.ShapeDtypeStruct((M, C), F32),
    )(x2d, w9, b2)


def _pool_body(x_ref, out):
    out[...] = jnp.maximum(x_ref[:, 0, :], x_ref[:, 1, :])


def _pool_pairs(x3):
    G, _, L = x3.shape
    gb = min(G, max(8, (2 * 1024 * 1024) // (2 * L * 4)) // 8 * 8)
    while G % gb:
        gb //= 2
    return pl.pallas_call(
        _pool_body,
        grid=(G // gb,),
        in_specs=[pl.BlockSpec((gb, 2, L), lambda m: (m, 0, 0))],
        out_specs=pl.BlockSpec((gb, L), lambda m: (m, 0)),
        out_shape=jax.ShapeDtypeStruct((G, L), F32),
    )(x3)


def _maxpool2(x2d, B, H, W, C):
    xh = x2d.reshape(B * (H // 2), 2, W * C)
    x2 = _pool_pairs(xh)
    xw = x2.reshape(B * (H // 2) * (W // 2), 2, C)
    x3 = _pool_pairs(xw)
    return x3


def _c4_body(HW, W, N, nvalid, x_ref, w_ref, b_ref, out):
    kb, _, C = x_ref.shape
    xf = x_ref[...].reshape(kb * HW, C)
    acc = jnp.zeros((kb * HW, N), F32)
    for di in range(3):
        for dj in range(3):
            s = di * W + dj
            t = di * 3 + dj
            if s:
                xs = jnp.concatenate(
                    [xf[s:], jnp.zeros((s, C), F32)], axis=0)
            else:
                xs = xf
            acc = acc + lax.dot_general(
                xs, w_ref[t * C:(t + 1) * C, :], (((1,), (0,)), ((), ())),
                preferred_element_type=F32)
    y = acc.reshape(kb, HW, N)
    rl = lax.broadcasted_iota(jnp.int32, (1, HW, 1), 1)
    m = ((rl // W < HW // W - 2) & (rl % W < W - 2)).astype(F32)
    ssum = jnp.sum(y * m, axis=1)
    out[...] = ssum * (1.0 / nvalid) + b_ref[...]


def _c4_mean(x3, wconv, bias, H, W):
    B, HW, C = x3.shape
    N = wconv.shape[0]
    w9 = jnp.transpose(wconv, (2, 3, 1, 0)).reshape(9 * C, N)
    b2 = bias.reshape(1, N)
    kb = min(B, 64)
    assert B % kb == 0
    nvalid = (H - 2) * (W - 2)
    return pl.pallas_call(
        functools.partial(_c4_body, HW, W, N, nvalid),
        grid=(B // kb,),
        in_specs=[pl.BlockSpec((kb, HW, C), lambda m: (m, 0, 0)),
                  pl.BlockSpec((9 * C, N), lambda m: (0, 0)),
                  pl.BlockSpec((1, N), lambda m: (0, 0))],
        out_specs=pl.BlockSpec((kb, N), lambda m: (m, 0)),
        out_shape=jax.ShapeDtypeStruct((B, N), F32),
    )(x3, w9, b2)


def _head_body(x_ref, w0, b0, w1, b1, out):
    h = lax.dot_general(x_ref[...], w0[...], (((1,), (1,)), ((), ())),
                        preferred_element_type=F32) + b0[...]
    h = jnp.maximum(h, 0.0)
    y = lax.dot_general(h, w1[...], (((1,), (1,)), ((), ())),
                        preferred_element_type=F32) + b1[...]
    y = jnp.maximum(y, 0.0)
    m = jnp.max(y, axis=1, keepdims=True)
    e = jnp.exp(y - m)
    out[...] = (y - m) - jnp.log(jnp.sum(e, axis=1, keepdims=True))


def _head(x, w0, b0, w1, b1):
    M, K = x.shape
    N0 = w0.shape[0]
    N1 = w1.shape[0]
    bm = min(M, 1024)
    return pl.pallas_call(
        _head_body,
        grid=(M // bm,),
        in_specs=[pl.BlockSpec((bm, K), lambda m: (m, 0)),
                  pl.BlockSpec((N0, K), lambda m: (0, 0)),
                  pl.BlockSpec((1, N0), lambda m: (0, 0)),
                  pl.BlockSpec((N1, N0), lambda m: (0, 0)),
                  pl.BlockSpec((1, N1), lambda m: (0, 0))],
        out_specs=pl.BlockSpec((bm, N1), lambda m: (m, 0)),
        out_shape=jax.ShapeDtypeStruct((M, N1), F32),
    )(x, w0, b0.reshape(1, N0), w1, b1.reshape(1, N1))


def _recip_body(p_ref, out):
    c = jnp.sum(jnp.sum(p_ref[...], axis=0), axis=1, keepdims=True)
    out[...] = 1.0 / jnp.maximum(c, 1.0)


def _recip_counts(partial):
    W, V, L = partial.shape
    return pl.pallas_call(
        _recip_body,
        in_specs=[pl.BlockSpec((W, V, L), lambda: (0, 0, 0))],
        out_specs=pl.BlockSpec((V, 1), lambda: (0, 0)),
        out_shape=jax.ShapeDtypeStruct((V, 1), F32),
    )(partial)


def _combine_body(p_ref, r_ref, out):
    r = r_ref[...]
    parts = [(p_ref[0, ct] + p_ref[1, ct]) * r for ct in range(p_ref.shape[1])]
    out[...] = jnp.concatenate(parts, axis=1)


def _combine_mean(partial, recip):
    _, CT, V, dsub = partial.shape
    return pl.pallas_call(
        _combine_body,
        in_specs=[pl.BlockSpec((2, CT, V, dsub), lambda: (0, 0, 0, 0)),
                  pl.BlockSpec((V, 1), lambda: (0, 0))],
        out_specs=pl.BlockSpec((V, CT * dsub), lambda: (0, 0)),
        out_shape=jax.ShapeDtypeStruct((V, CT * dsub), F32),
    )(partial, recip)


_NW = 32


def _sc_gather(table, idx):
    V, D = table.shape
    B = idx.shape[0]
    bpw = B // _NW
    ch = min(128, bpw, max(16, 65536 // D))
    n_ch = bpw // ch
    assert bpw % ch == 0
    mesh = plsc.VectorSubcoreMesh(core_axis_name="c", subcore_axis_name="s")

    @functools.partial(
        pl.kernel, mesh=mesh,
        out_type=jax.ShapeDtypeStruct((B, D), F32),
        scratch_types=[pltpu.VMEM((ch,), jnp.int32),
                       pltpu.VMEM((ch, D), F32),
                       pltpu.SemaphoreType.DMA],
    )
    def k(table_hbm, idx_hbm, out_hbm, idx_v, rows_v, sem):
        wid = lax.axis_index("s") * 2 + lax.axis_index("c")
        base = wid * bpw
        for j in range(n_ch):
            off = base + j * ch
            pltpu.sync_copy(idx_hbm.at[pl.ds(off, ch)], idx_v)
            pltpu.async_copy(table_hbm.at[idx_v], rows_v, sem).wait()
            pltpu.sync_copy(rows_v, out_hbm.at[pl.ds(off, ch)])

    return k(table, idx)


def _sc_scatter_add(vals_t, idx, V):
    D, E = vals_t.shape
    dsub = D // 16
    assert dsub >= 16 and dsub % 16 == 0
    eh = E // 2
    ch_e = 128
    n_ch = eh // ch_e
    zeros = jnp.zeros((V * dsub,), F32)
    mesh = plsc.VectorSubcoreMesh(core_axis_name="c", subcore_axis_name="s")

    @functools.partial(
        pl.kernel, mesh=mesh,
        out_type=jax.ShapeDtypeStruct((2, 16, V * dsub), F32),
        scratch_types=[pltpu.VMEM((ch_e,), jnp.int32),
                       pltpu.VMEM((dsub, ch_e), F32),
                       pltpu.VMEM((V * dsub,), F32)],
        compiler_params=pltpu.CompilerParams(needs_layout_passes=False),
    )
    def k(vals_hbm, idx_hbm, zeros_hbm, out_hbm, idx_v, val_v, acc_v):
        c = lax.axis_index("c")
        s = lax.axis_index("s")
        wid = s * 2 + c
        rep = wid // 16
        ct = wid % 16
        pltpu.sync_copy(zeros_hbm, acc_v)
        lanes = lax.iota(jnp.int32, 16)
        for j in range(n_ch):
            off = rep * eh + j * ch_e
            pltpu.sync_copy(idx_hbm.at[pl.ds(off, ch_e)], idx_v)
            pltpu.sync_copy(
                vals_hbm.at[pl.ds(ct * dsub, dsub), pl.ds(off, ch_e)], val_v)

            def body(eg, _):
                rvec = idx_v[pl.ds(eg * 16, 16)]
                for l in range(16):
                    e = eg * 16 + l
                    rows = jnp.full((16,), rvec[l], jnp.int32)
                    es = jnp.full((16,), e, jnp.int32)
                    for g in range(dsub // 16):
                        v = plsc.load_gather(val_v, [g * 16 + lanes, es])
                        plsc.addupdate_scatter(
                            acc_v, [rows * dsub + g * 16 + lanes], v)
                return 0

            lax.fori_loop(0, ch_e // 16, body, 0)
        pltpu.sync_copy(acc_v, out_hbm.at[rep, ct])

    return k(vals_t, idx, zeros)


def _sc_counts(idx, E, V):
    epw = E // _NW
    zeros = jnp.zeros((V * 16,), F32)
    mesh = plsc.VectorSubcoreMesh(core_axis_name="c", subcore_axis_name="s")

    @functools.partial(
        pl.kernel, mesh=mesh,
        out_type=jax.ShapeDtypeStruct((_NW, V * 16), F32),
        scratch_types=[pltpu.VMEM((epw,), jnp.int32),
                       pltpu.VMEM((V * 16,), F32)],
        compiler_params=pltpu.CompilerParams(needs_layout_passes=False),
    )
    def k(idx_hbm, zeros_hbm, out_hbm, idx_v, acc_v):
        c = lax.axis_index("c")
        s = lax.axis_index("s")
        wid = s * 2 + c
        pltpu.sync_copy(zeros_hbm, acc_v)
        pltpu.sync_copy(idx_hbm.at[pl.ds(wid * epw, epw)], idx_v)
        lanes = lax.iota(jnp.int32, 16)
        one = jnp.ones((16,), F32)

        def body(g, _):
            rows = idx_v[pl.ds(g * 16, 16)]
            plsc.addupdate_scatter(acc_v, [rows * 16 + lanes], one)
            return 0

        lax.fori_loop(0, epw // 16, body, 0)
        pltpu.sync_copy(acc_v, out_hbm.at[wid])

    return k(idx, zeros)


def _cnn(p, imgs, c_mid, n_pools):
    B = imgs.shape[0]
    x = _dwpw1(imgs, p["dw1"], p["pw1"], c_mid)
    x = _dw(x, p["dw2"][0], p["dw2"][1], 16, 16)
    x = _mm([x], [p["pw2"][0].reshape(c_mid, c_mid)], p["pw2"][1])
    x = _maxpool2(x, B, 16, 16, c_mid)
    x = _dw(x, p["dw3"][0], p["dw3"][1], 8, 8)
    c_out = p["pw3"][0].shape[0]
    x = _mm([x], [p["pw3"][0].reshape(c_out, c_mid)], p["pw3"][1])
    hw = 8
    if n_pools == 2:
        x = _maxpool2(x, B, 8, 8, c_out)
        hw = 4
    x = _c4_mean(x.reshape(B, hw * hw, c_out), p["c4"][0], p["c4"][1], hw, hw)
    return x


def _split_cols(W, widths):
    out = []
    o = 0
    for w in widths:
        out.append(W[:, o:o + w])
        o += w
    return out


def kernel(x, edge_attr, node_image_regions, edge_image_regions, edge_index,
           params):
    p = params
    row = edge_index[0].astype(jnp.int32)
    idx_all = edge_index.astype(jnp.int32).reshape(-1)
    n_nodes = x.shape[0]
    n_edges = edge_attr.shape[0]

    nv = _cnn(p["node_cnn"], node_image_regions, 64, 1)
    ev = _cnn(p["edge_cnn"], edge_image_regions, 128, 2)

    wj = _split_cols(p["node_join"][0], [x.shape[1], 256])
    xc = _mm([x, nv], wj, p["node_join"][1])
    wj = _split_cols(p["edge_join"][0], [edge_attr.shape[1], 256])
    ea = _mm([edge_attr, ev], wj, p["edge_join"][1])

    cnt_partial = _sc_counts(row, n_edges, n_nodes).reshape(_NW, n_nodes, 16)
    recip = _recip_counts(cnt_partial)

    dims = [(256, 256, 256, 512, 512), (512, 512, 512, 1024, 1024),
            (1024, 1024, 1024, 512, 512), (512, 512, 512, 256, 256)]
    for i, (inn, ine, hid, outn, oute) in enumerate(dims):
        pe = p["l%d_edge" % (i + 1)]
        pn = p["l%d_node" % (i + 1)]
        g = _sc_gather(xc, idx_all)
        src, dst = g[:n_edges], g[n_edges:]
        ws = _split_cols(pe["mlp0"][0], [inn, inn, ine])
        h = _mm([src, dst, ea], ws, pe["mlp0"][1], relu=True)
        h = _mm([h], [pe["mlp1"][0]], pe["mlp1"][1])
        ws = _split_cols(pe["res"][0], [oute, ine])
        ea = _mm([h, ea], ws, pe["res"][1])
        ws = _split_cols(pn["mlp1_0"][0], [inn, oute])
        h = _mm([dst, ea], ws, pn["mlp1_0"][1], relu=True)
        ht = _mm([h], [pn["mlp1_1"][0]], pn["mlp1_1"][1], t_out=True)
        part = _sc_scatter_add(ht, row, n_nodes)
        part = part.reshape(2, 16, n_nodes, outn // 16)
        agg = _combine_mean(part, recip)
        ws = _split_cols(pn["mlp2_0"][0], [inn, outn])
        h = _mm([xc, agg], ws, pn["mlp2_0"][1], relu=True)
        h = _mm([h], [pn["mlp2_1"][0]], pn["mlp2_1"][1])
        ws = _split_cols(pn["res"][0], [outn, inn])
        xc = _mm([h, xc], ws, pn["res"][1])

    xn = _head(xc, p["node_cls0"][0], p["node_cls0"][1],
               p["node_cls1"][0], p["node_cls1"][1])
    xe = _head(ea, p["edge_cls0"][0], p["edge_cls0"][1],
               p["edge_cls1"][0], p["edge_cls1"][1])
    return (xn, xe)
